# baseline (speedup 1.0000x reference)
import functools

import jax
import jax.numpy as jnp
from jax import lax
from jax.experimental import pallas as pl
from jax.experimental.pallas import tpu as pltpu
from jax.experimental.pallas import tpu_sc as plsc

B = 16384
D = 50
DU = 64
NW = 32
ROWS_PER_W = B // NW
CHUNK = 128
NCHUNK = ROWS_PER_W // CHUNK

BLK = 1024
F_FT = 768


def _gather_body(tab, idx_hbm, out, idx_v, rows_v, sem):
    c = lax.axis_index("c")
    s = lax.axis_index("s")
    wid = s * 2 + c
    base = wid * ROWS_PER_W
    pltpu.sync_copy(idx_hbm.at[wid], idx_v)
    descs = []
    for j in range(NCHUNK):
        descs.append(pltpu.async_copy(
            tab.at[idx_v.at[j]], rows_v.at[pl.ds(j * CHUNK, CHUNK)], sem))
    for dsc in descs:
        dsc.wait()
    pltpu.sync_copy(rows_v, out.at[pl.ds(base, ROWS_PER_W)])


def _sc_gather(table, idx, width, dtype):
    mesh = plsc.VectorSubcoreMesh(core_axis_name="c", subcore_axis_name="s")
    return pl.kernel(
        _gather_body,
        out_type=jax.ShapeDtypeStruct((B, width), dtype),
        mesh=mesh,
        scratch_types=[
            pltpu.VMEM((NCHUNK, CHUNK), jnp.int32),
            pltpu.VMEM((ROWS_PER_W, width), dtype),
            pltpu.SemaphoreType.DMA,
        ],
        compiler_params=pltpu.CompilerParams(use_tc_tiling_on_sc=False),
    )(table, idx)


def _mlp_body(ft, u_e, b_e, a_e, p_e, cat, scal,
              w1t, w1u, w1b, w1a, w1p, w1c, w1s, b1, w2, b2, w3, b3,
              out_ref):
    f32 = jnp.float32
    acc = jnp.dot(ft[...], w1t[...], preferred_element_type=f32)
    acc += jnp.dot(u_e[...].astype(f32), w1u[...], preferred_element_type=f32)
    acc += jnp.dot(b_e[...], w1b[...], preferred_element_type=f32)
    acc += jnp.dot(a_e[...], w1a[...], preferred_element_type=f32)
    acc += jnp.dot(p_e[...], w1p[...], preferred_element_type=f32)
    acc += jnp.dot(cat[...], w1c[...], preferred_element_type=f32)
    acc += jnp.dot(scal[...], w1s[...], preferred_element_type=f32)
    h1 = jnp.maximum(acc + b1[...], 0.0)
    h2 = jnp.maximum(jnp.dot(h1, w2[...], preferred_element_type=f32) + b2[...], 0.0)
    out = jnp.sum(h2 * w3[...], axis=1, keepdims=True) + b3[0, 0]
    out_ref[...] = out


def _mlp(ft, u_e, b_e, a_e, p_e, cat, scal,
         w1t, w1u, w1b, w1a, w1p, w1c, w1s, b1, w2, b2, w3, b3):
    grid = (B // BLK,)
    row = lambda i: (i, 0)
    const = lambda i: (0, 0)
    in_specs = [
        pl.BlockSpec((BLK, F_FT), row),
        pl.BlockSpec((BLK, DU), row),
        pl.BlockSpec((BLK, D), row),
        pl.BlockSpec((BLK, D), row),
        pl.BlockSpec((BLK, D), row),
        pl.BlockSpec((BLK, 9), row),
        pl.BlockSpec((BLK, 4), row),
        pl.BlockSpec((F_FT, 128), const),
        pl.BlockSpec((DU, 128), const),
        pl.BlockSpec((D, 128), const),
        pl.BlockSpec((D, 128), const),
        pl.BlockSpec((D, 128), const),
        pl.BlockSpec((9, 128), const),
        pl.BlockSpec((4, 128), const),
        pl.BlockSpec((1, 128), const),
        pl.BlockSpec((128, 64), const),
        pl.BlockSpec((1, 64), const),
        pl.BlockSpec((1, 64), const),
        pl.BlockSpec((1, 1), const),
    ]
    return pl.pallas_call(
        _mlp_body,
        grid=grid,
        in_specs=in_specs,
        out_specs=pl.BlockSpec((BLK, 1), row),
        out_shape=jax.ShapeDtypeStruct((B, 1), jnp.float32),
    )(ft, u_e, b_e, a_e, p_e, cat, scal,
      w1t, w1u, w1b, w1a, w1p, w1c, w1s, b1, w2, b2, w3, b3)


def _prep_idx(ids):
    return ids.astype(jnp.int32).reshape(NW, NCHUNK, CHUNK)


def kernel(user_id, book_id, author_label, category_label, publisher_label,
           page_count, average_rating, ratings_count, published_year,
           full_text_embeddings, user_table, book_table, author_table,
           publisher_table, W1, b1, W2, b2, W3, b3):
    user16 = jnp.pad(user_table, ((0, 0), (0, DU - D))).astype(jnp.bfloat16)
    u_e = _sc_gather(user16, _prep_idx(user_id), DU, jnp.bfloat16)
    b_e = _sc_gather(book_table, _prep_idx(book_id), D, jnp.float32)
    a_e = _sc_gather(author_table, _prep_idx(author_label), D, jnp.float32)
    p_e = _sc_gather(publisher_table, _prep_idx(publisher_label), D,
                     jnp.float32)

    scal = jnp.stack([page_count, average_rating, ratings_count,
                      published_year], axis=1)

    W1T = W1.T
    w1u = jnp.zeros((DU, 128), jnp.float32).at[0:D].set(W1T[0:50])
    w1b = W1T[50:100]
    w1a = W1T[100:150]
    w1c = W1T[150:159]
    w1p = W1T[159:209]
    w1s = W1T[209:213]
    w1t = W1T[213:981]

    out = _mlp(full_text_embeddings, u_e, b_e, a_e, p_e,
               category_label, scal,
               w1t, w1u, w1b, w1a, w1p, w1c, w1s,
               b1.reshape(1, 128), W2.T, b2.reshape(1, 64),
               W3.reshape(1, 64), b3.reshape(1, 1))
    return out.reshape(B)
